# Initial kernel scaffold; baseline (speedup 1.0000x reference)
import jax
import jax.numpy as jnp
from jax.experimental import pallas as pl


def kernel(x, edge_index, batch, W1, b1, W2, b2, Wm1, bm1, Wm2, bm2):
    raise NotImplementedError("write your pallas kernel here")


# speedup vs baseline: 12.4059x; 12.4059x over previous
import functools

import jax
import jax.numpy as jnp
from jax import lax
from jax.experimental import pallas as pl
from jax.experimental.pallas import tpu as pltpu
from jax.experimental.pallas import tpu_sc as plsc

_N = 50000
_E = 800000
_G = 64
_D = 128
_ND = 112
_H = 64
_Q = 8
_NQ = _H // _Q

_CHUNK = 128
_NCHUNK = 6400
_EPAD = _NCHUNK * _CHUNK
_CPT = _NCHUNK // 16
_CPT_DEG = _NCHUNK // 32
_NPAD = 50048
_RPT = _NPAD // 16
_NB = 4
_SB = 100
_NSB = _CPT // _SB

_BR = 1000
_NBLK = _N // _BR


@functools.cache
def _get_mesh():
    return plsc.VectorSubcoreMesh(core_axis_name="c", subcore_axis_name="s",
                                  num_cores=2, num_subcores=16)


def _deg_body(dst2d, ones_h, zer1_h, out, idxs, ones_v, zbuf, acc):
    c = lax.axis_index("c")
    s = lax.axis_index("s")
    wid = c * 16 + s
    pltpu.sync_copy(dst2d.at[pl.ds(wid * _CPT_DEG, _CPT_DEG)], idxs)
    pltpu.sync_copy(ones_h, ones_v)
    pltpu.sync_copy(zer1_h, zbuf)
    pltpu.sync_copy(zbuf, acc.at[pl.ds(s * _RPT, _RPT)])
    plsc.subcore_barrier()

    def body(j, carry):
        pltpu.sync_copy(ones_v, acc.at[idxs.at[j]], add=True)
        return carry

    lax.fori_loop(0, _CPT_DEG, body, 0)
    plsc.subcore_barrier()
    pltpu.sync_copy(acc.at[pl.ds(s * _RPT, _RPT)], zbuf)
    pltpu.sync_copy(zbuf, out.at[pl.ds(c * _NPAD + s * _RPT, _RPT)])


def _deg_call(dst2d, ones_h, zer1_h):
    return pl.kernel(
        _deg_body,
        out_type=jax.ShapeDtypeStruct((2 * _NPAD,), jnp.float32),
        mesh=_get_mesh(),
        scratch_types=[
            pltpu.VMEM((_CPT_DEG, _CHUNK), jnp.int32),
            pltpu.VMEM((_CHUNK,), jnp.float32),
            pltpu.VMEM((_RPT,), jnp.float32),
            pltpu.VMEM_SHARED((_NPAD,), jnp.float32),
        ],
    )(dst2d, ones_h, zer1_h)


def _scat_body(src2d, dst2d, ta, tb, zrows_h, out,
               srcs, dsts, r0, r1, r2, r3, acc, s0, s1, s2, s3):
    c = lax.axis_index("c")
    s = lax.axis_index("s")
    rows = [r0, r1, r2, r3]
    sems = [s0, s1, s2, s3]
    pltpu.sync_copy(zrows_h, r0)
    for k in range(_RPT // _CHUNK):
        pltpu.sync_copy(r0, acc.at[pl.ds(s * _RPT + k * _CHUNK, _CHUNK)])
    _rem = _RPT % _CHUNK
    pltpu.sync_copy(r0.at[pl.ds(0, _rem)],
                    acc.at[pl.ds(s * _RPT + (_RPT // _CHUNK) * _CHUNK, _rem)])
    plsc.subcore_barrier()

    def start_gather(j, b):
        @pl.when(c == 0)
        def _():
            pltpu.async_copy(ta.at[srcs.at[j]], rows[b], sems[b])

        @pl.when(c == 1)
        def _():
            pltpu.async_copy(tb.at[srcs.at[j]], rows[b], sems[b])

    def wait_gather(b):
        pltpu.make_async_copy(ta.at[srcs.at[0]], rows[b], sems[b]).wait()

    def body(jj, carry):
        j = jj * _NB
        for b in range(_NB):
            wait_gather(b)
            pltpu.sync_copy(rows[b], acc.at[dsts.at[j + b]], add=True)

            @pl.when(j + b + _NB < _SB)
            def _():
                start_gather(j + b + _NB, b)
        return carry

    for sb in range(_NSB):
        base = s * _CPT + sb * _SB
        pltpu.sync_copy(src2d.at[pl.ds(base, _SB)], srcs)
        pltpu.sync_copy(dst2d.at[pl.ds(base, _SB)], dsts)
        for b in range(_NB):
            start_gather(b, b)
        lax.fori_loop(0, _SB // _NB, body, 0)
    plsc.subcore_barrier()
    obase = c * _NPAD + s * _RPT
    pltpu.sync_copy(acc.at[pl.ds(s * _RPT, _RPT)],
                    out.at[pl.ds(obase, _RPT)])


def _scat_call(src2d, dst2d, ta, tb, zrows_h):
    return pl.kernel(
        _scat_body,
        out_type=jax.ShapeDtypeStruct((2 * _NPAD, _Q), jnp.float32),
        mesh=_get_mesh(),
        compiler_params=pltpu.CompilerParams(use_tc_tiling_on_sc=False),
        scratch_types=[
            pltpu.VMEM((_SB, _CHUNK), jnp.int32),
            pltpu.VMEM((_SB, _CHUNK), jnp.int32),
            pltpu.VMEM((_CHUNK, _Q), jnp.float32),
            pltpu.VMEM((_CHUNK, _Q), jnp.float32),
            pltpu.VMEM((_CHUNK, _Q), jnp.float32),
            pltpu.VMEM((_CHUNK, _Q), jnp.float32),
            pltpu.VMEM_SHARED((_NPAD, _Q), jnp.float32),
            pltpu.SemaphoreType.DMA,
            pltpu.SemaphoreType.DMA,
            pltpu.SemaphoreType.DMA,
            pltpu.SemaphoreType.DMA,
        ],
    )(src2d, dst2d, ta, tb, zrows_h)


def _qspec():
    return pl.BlockSpec((_BR, _Q), lambda i: (i, 0))


def _vspec():
    return pl.BlockSpec((_BR, 1), lambda i: (i, 0))


def _pre_body(*refs):
    x_ref, d0_ref, d1_ref, b_ref, w1_ref = refs[:5]
    h_refs = refs[5:5 + _NQ]
    dinv_ref, gsum_ref, cnt_ref = refs[5 + _NQ:8 + _NQ]
    i = pl.program_id(0)
    deg = 1.0 + d0_ref[...] + d1_ref[...]
    dinv = lax.rsqrt(deg)
    dinv_ref[...] = dinv
    xb = x_ref[...]
    h = jnp.dot(xb[:, :_ND], w1_ref[...], preferred_element_type=jnp.float32)
    hs = h * dinv
    for q in range(_NQ):
        h_refs[q][...] = hs[:, q * _Q:(q + 1) * _Q]
    onehot = (b_ref[...] == lax.broadcasted_iota(jnp.int32, (1, _G), 1)
              ).astype(jnp.float32)
    g = lax.dot_general(onehot, xb[:, _ND:], (((0,), (0,)), ((), ())),
                        preferred_element_type=jnp.float32)
    cntc = lax.dot_general(onehot, jnp.ones((_BR, 1), jnp.float32),
                           (((0,), (0,)), ((), ())),
                           preferred_element_type=jnp.float32)

    @pl.when(i == 0)
    def _():
        gsum_ref[...] = jnp.zeros_like(gsum_ref)
        cnt_ref[...] = jnp.zeros_like(cnt_ref)

    gsum_ref[...] += g
    cnt_ref[...] += cntc


def _pre_call(x, d0, d1, batch2, w1):
    return pl.pallas_call(
        _pre_body,
        grid=(_NBLK,),
        in_specs=[
            pl.BlockSpec((_BR, _D), lambda i: (i, 0)),
            _vspec(), _vspec(), _vspec(),
            pl.BlockSpec((_ND, _H), lambda i: (0, 0)),
        ],
        out_specs=[_qspec() for _ in range(_NQ)] + [
            _vspec(),
            pl.BlockSpec((_G, _D - _ND), lambda i: (0, 0)),
            pl.BlockSpec((_G, 1), lambda i: (0, 0)),
        ],
        out_shape=[jax.ShapeDtypeStruct((_N, _Q), jnp.float32)
                   for _ in range(_NQ)] + [
            jax.ShapeDtypeStruct((_N, 1), jnp.float32),
            jax.ShapeDtypeStruct((_G, _D - _ND), jnp.float32),
            jax.ShapeDtypeStruct((_G, 1), jnp.float32),
        ],
    )(x, d0, d1, batch2, w1)


def _mid_body(*refs):
    a_refs = refs[0:_NQ]
    h_refs = refs[_NQ:2 * _NQ]
    dinv_ref, w2_ref, b1_ref = refs[2 * _NQ:2 * _NQ + 3]
    o_refs = refs[2 * _NQ + 3:]
    scat = jnp.concatenate([r[...] for r in a_refs], axis=1)
    hs1 = jnp.concatenate([r[...] for r in h_refs], axis=1)
    dinv = dinv_ref[...]
    r1 = jnp.maximum((scat + hs1) * dinv + b1_ref[...], 0.0)
    h2 = jnp.dot(r1, w2_ref[...], preferred_element_type=jnp.float32)
    hs2 = h2 * dinv
    for q in range(_NQ):
        o_refs[q][...] = hs2[:, q * _Q:(q + 1) * _Q]


def _mid_call(accq, hsq, dinv, w2, b1r):
    return pl.pallas_call(
        _mid_body,
        grid=(_NBLK,),
        in_specs=[_qspec() for _ in range(2 * _NQ)] + [
            _vspec(),
            pl.BlockSpec((_H, _H), lambda i: (0, 0)),
            pl.BlockSpec((1, _H), lambda i: (0, 0)),
        ],
        out_specs=[_qspec() for _ in range(_NQ)],
        out_shape=[jax.ShapeDtypeStruct((_N, _Q), jnp.float32)] * _NQ,
    )(*accq, *hsq, dinv, w2, b1r)


def _post_body(*refs):
    a_refs = refs[0:_NQ]
    h_refs = refs[_NQ:2 * _NQ]
    (dinv_ref, b_ref, b2_ref, gsum_ref, cnt_ref, wm1_ref, bm1_ref, wm2_ref,
     bm2_ref, out_ref, psum_ref) = refs[2 * _NQ:]
    i = pl.program_id(0)
    scat = jnp.concatenate([r[...] for r in a_refs], axis=1)
    hs2 = jnp.concatenate([r[...] for r in h_refs], axis=1)
    h2 = (scat + hs2) * dinv_ref[...] + b2_ref[...]
    onehot = (b_ref[...] == lax.broadcasted_iota(jnp.int32, (1, _G), 1)
              ).astype(jnp.float32)
    p = lax.dot_general(onehot, h2, (((0,), (0,)), ((), ())),
                        preferred_element_type=jnp.float32)

    @pl.when(i == 0)
    def _():
        psum_ref[...] = jnp.zeros_like(psum_ref)

    psum_ref[...] += p

    @pl.when(i == _NBLK - 1)
    def _():
        cnt = jnp.maximum(cnt_ref[...], 1.0)
        pooled = psum_ref[...] / cnt
        gpool = gsum_ref[...] / cnt
        z = jnp.concatenate([pooled, gpool], axis=1)
        z1 = jnp.maximum(
            jnp.dot(z, wm1_ref[...], preferred_element_type=jnp.float32)
            + bm1_ref[...], 0.0)
        out_ref[...] = (
            jnp.dot(z1, wm2_ref[...], preferred_element_type=jnp.float32)
            + bm2_ref[...])


def _post_call(accq, hsq, dinv, batch2, b2r, gsum, cnt, wm1, bm1r, wm2, bm2r):
    zdim = _H + (_D - _ND)
    return pl.pallas_call(
        _post_body,
        grid=(_NBLK,),
        in_specs=[_qspec() for _ in range(2 * _NQ)] + [
            _vspec(), _vspec(),
            pl.BlockSpec((1, _H), lambda i: (0, 0)),
            pl.BlockSpec((_G, _D - _ND), lambda i: (0, 0)),
            pl.BlockSpec((_G, 1), lambda i: (0, 0)),
            pl.BlockSpec((zdim, _H), lambda i: (0, 0)),
            pl.BlockSpec((1, _H), lambda i: (0, 0)),
            pl.BlockSpec((_H, 2), lambda i: (0, 0)),
            pl.BlockSpec((1, 2), lambda i: (0, 0)),
        ],
        out_specs=pl.BlockSpec((_G, 2), lambda i: (0, 0)),
        out_shape=jax.ShapeDtypeStruct((_G, 2), jnp.float32),
        scratch_shapes=[pltpu.VMEM((_G, _H), jnp.float32)],
    )(*accq, *hsq, dinv, batch2, b2r, gsum, cnt, wm1, bm1r, wm2, bm2r)


def _conv_scatter(src2d, dst2d, hsq, zrows_h):
    accq = []
    for i in range(_NQ // 2):
        a = _scat_call(src2d, dst2d, hsq[2 * i], hsq[2 * i + 1], zrows_h)
        a = a.reshape(2, _NPAD, _Q)
        accq.append(a[0, :_N])
        accq.append(a[1, :_N])
    return accq


def kernel(x, edge_index, batch, W1, b1, W2, b2, Wm1, bm1, Wm2, bm2):
    src = edge_index[0]
    dst = edge_index[1]
    npad = _EPAD - _E
    padi = lax.iota(jnp.int32, npad) % 48
    src_p = jnp.concatenate([src, padi])
    dst_p = jnp.concatenate([dst, _N + padi])
    src2d = src_p.reshape(_NCHUNK, _CHUNK)
    dst2d = dst_p.reshape(_NCHUNK, _CHUNK)

    ones_h = jnp.ones((_CHUNK,), jnp.float32)
    zer1_h = jnp.zeros((_RPT,), jnp.float32)
    zrows_h = jnp.zeros((_CHUNK, _Q), jnp.float32)

    degp = _deg_call(dst2d, ones_h, zer1_h).reshape(2, _NPAD)
    d0 = degp[0, :_N].reshape(_N, 1)
    d1 = degp[1, :_N].reshape(_N, 1)
    batch2 = batch.reshape(_N, 1)

    pre = _pre_call(x, d0, d1, batch2, W1)
    hs1q = list(pre[:_NQ])
    dinv, gsum, cnt = pre[_NQ:]

    acc1q = _conv_scatter(src2d, dst2d, hs1q, zrows_h)
    hs2q = list(_mid_call(acc1q, hs1q, dinv, W2, b1.reshape(1, _H)))

    acc2q = _conv_scatter(src2d, dst2d, hs2q, zrows_h)
    out = _post_call(acc2q, hs2q, dinv, batch2, b2.reshape(1, _H), gsum, cnt,
                     Wm1, bm1.reshape(1, _H), Wm2, bm2.reshape(1, 2))
    return out
